# Initial kernel scaffold; baseline (speedup 1.0000x reference)
import numpy as np
import jax
import jax.numpy as jnp
from jax.experimental import pallas as pl
from jax.experimental.pallas import tpu as pltpu

_NUM_CLASSES = 80
_INPUT_SIZE = 608.0
_ANCHORS = np.array(
    [[10, 13], [16, 30], [33, 23], [30, 61], [62, 45], [59, 119],
     [116, 90], [156, 198], [373, 326]], dtype=np.float32)
_MASKS = [[6, 7, 8], [3, 4, 5], [0, 1, 2]]
_TOPK = 1000
_MAX_DET = 100
_IOU_TH = 0.5
_GRIDS = (19, 38, 76)
_N = 3 * sum(g * g for g in _GRIDS)
_ROWS = (_N + 127) // 128
_NPAD = _ROWS * 128
_C = 5 + _NUM_CLASSES


def _build_consts() -> np.ndarray:
    gxs, gys, gs, aws, ahs = [], [], [], [], []
    for G, m in zip(_GRIDS, _MASKS):
        cell = np.arange(G * G)
        gx = (cell % G).astype(np.float32)
        gy = (cell // G).astype(np.float32)
        gxs.append(np.repeat(gx, 3))
        gys.append(np.repeat(gy, 3))
        gs.append(np.full(G * G * 3, float(G), np.float32))
        anc = _ANCHORS[np.array(m)]
        aws.append(np.tile(anc[:, 0], G * G))
        ahs.append(np.tile(anc[:, 1], G * G))
    out = np.stack([np.concatenate(v) for v in (gxs, gys, gs, aws, ahs)])
    pad = np.ones((5, _NPAD - _N), np.float32)
    out = np.concatenate([out, pad], axis=1)
    return out.reshape(5, _ROWS, 128)


def _extract(arr, onehot):
    return jnp.sum(jnp.where(onehot, arr, 0.0))


def _yolo_kernel(const_ref, x_ref, out_ref):
    x = x_ref[0]
    gx = const_ref[0]
    gy = const_ref[1]
    gv = const_ref[2]
    aw = const_ref[3]
    ah = const_ref[4]

    obj = jax.nn.sigmoid(x[4])
    cls_s = jax.nn.sigmoid(x[5:])
    prod = cls_s * obj[None]
    conf = jnp.max(prod, axis=0)
    riota = jax.lax.broadcasted_iota(jnp.float32, prod.shape, 0)
    cls = jnp.min(jnp.where(prod == conf[None], riota, 1e9), axis=0)

    sx = jax.nn.sigmoid(x[0])
    sy = jax.nn.sigmoid(x[1])
    w = jnp.exp(jnp.clip(x[2], -10.0, 10.0)) * aw / _INPUT_SIZE
    h = jnp.exp(jnp.clip(x[3], -10.0, 10.0)) * ah / _INPUT_SIZE
    xc = (sx + gx) / gv
    yc = (sy + gy) / gv
    x1 = xc - w * 0.5
    y1 = yc - h * 0.5
    x2 = xc + w * 0.5
    y2 = yc + h * 0.5

    rr = jax.lax.broadcasted_iota(jnp.int32, (_ROWS, 128), 0)
    cc = jax.lax.broadcasted_iota(jnp.int32, (_ROWS, 128), 1)
    n = rr * 128 + cc
    conf = jnp.where(n < _N, conf, -1.0)

    bits = jax.lax.bitcast_convert_type(conf, jnp.int32)

    def bs_body(_, carry):
        lo, hi = carry
        mid = (lo + hi) // 2
        cnt = jnp.sum((bits >= mid).astype(jnp.int32))
        return (jnp.where(cnt >= _TOPK, mid, lo),
                jnp.where(cnt >= _TOPK, hi, mid))

    lo, hi = jax.lax.fori_loop(
        0, 31, bs_body, (jnp.int32(0), jnp.int32(0x40000000)))
    vbits = lo
    cnt_gt = jnp.sum((bits > vbits).astype(jnp.int32))
    need = _TOPK - cnt_gt
    eq = bits == vbits

    def ts_body(_, carry):
        lo_t, hi_t = carry
        mid = (lo_t + hi_t) // 2
        cnt = jnp.sum((eq & (n <= mid)).astype(jnp.int32))
        return (jnp.where(cnt >= need, lo_t, mid),
                jnp.where(cnt >= need, mid, hi_t))

    _, t_idx = jax.lax.fori_loop(
        0, 16, ts_body, (jnp.int32(-1), jnp.int32(_NPAD)))
    active0 = ((bits > vbits) | (eq & (n <= t_idx))).astype(jnp.float32)

    maxall = jnp.max(conf)
    j0 = jnp.min(jnp.where(conf == maxall, n, _NPAD))
    oh0 = n == j0
    t1x1 = _extract(x1, oh0)
    t1y1 = _extract(y1, oh0)
    t1x2 = _extract(x2, oh0)
    t1y2 = _extract(y2, oh0)
    t1cl = _extract(cls, oh0)

    area2 = jnp.clip(x2 - x1, 0.0) * jnp.clip(y2 - y1, 0.0)
    lane = jax.lax.broadcasted_iota(jnp.int32, (1, 128), 1)

    def nms_body(i, active):
        masked = jnp.where(active > 0, conf, -1e9)
        mv = jnp.max(masked)
        j = jnp.min(jnp.where(masked == mv, n, _NPAD))
        oh = n == j
        bx1 = _extract(x1, oh)
        by1 = _extract(y1, oh)
        bx2 = _extract(x2, oh)
        by2 = _extract(y2, oh)
        bcl = _extract(cls, oh)
        ex = mv <= -1e8

        fx1 = jnp.where(ex, t1x1, bx1)
        fy1 = jnp.where(ex, t1y1, by1)
        fx2 = jnp.where(ex, t1x2, bx2)
        fy2 = jnp.where(ex, t1y2, by2)
        fcl = jnp.where(ex, t1cl, bcl)
        fsc = jnp.where(ex, 0.0, mv)
        row = jnp.where(lane == 0, fx1,
              jnp.where(lane == 1, fy1,
              jnp.where(lane == 2, fx2,
              jnp.where(lane == 3, fy2,
              jnp.where(lane == 4, fsc,
              jnp.where(lane == 5, fcl, 0.0))))))
        out_ref[0, pl.ds(i, 1), :] = row

        ix1 = jnp.maximum(bx1, x1)
        iy1 = jnp.maximum(by1, y1)
        ix2 = jnp.minimum(bx2, x2)
        iy2 = jnp.minimum(by2, y2)
        inter = jnp.clip(ix2 - ix1, 0.0) * jnp.clip(iy2 - iy1, 0.0)
        area1 = jnp.clip(bx2 - bx1, 0.0) * jnp.clip(by2 - by1, 0.0)
        iou = inter / (area1 + area2 - inter + 1e-9)
        active = jnp.where(iou > _IOU_TH, 0.0, active)
        active = jnp.where(oh, 0.0, active)
        return active

    jax.lax.fori_loop(0, _MAX_DET, nms_body, active0)


def kernel(inputs_0, inputs_1, inputs_2):
    B = inputs_0.shape[0]
    parts = []
    for x in (inputs_0, inputs_1, inputs_2):
        G = x.shape[1]
        parts.append(jnp.transpose(x.reshape(B, G * G * 3, _C), (0, 2, 1)))
    full = jnp.concatenate(parts, axis=2)
    full = jnp.pad(full, ((0, 0), (0, 0), (0, _NPAD - _N)))
    full = full.reshape(B, _C, _ROWS, 128)
    consts = jnp.asarray(_build_consts())

    out = pl.pallas_call(
        _yolo_kernel,
        grid=(B,),
        in_specs=[
            pl.BlockSpec((5, _ROWS, 128), lambda b: (0, 0, 0)),
            pl.BlockSpec((1, _C, _ROWS, 128), lambda b: (b, 0, 0, 0)),
        ],
        out_specs=pl.BlockSpec((1, _MAX_DET, 128), lambda b: (b, 0, 0)),
        out_shape=jax.ShapeDtypeStruct((B, _MAX_DET, 128), jnp.float32),
        compiler_params=pltpu.CompilerParams(
            dimension_semantics=("arbitrary",)),
    )(consts, full)
    return out[:, :, :6]


# speedup vs baseline: 3.8252x; 3.8252x over previous
import numpy as np
import jax
import jax.numpy as jnp
from jax.experimental import pallas as pl
from jax.experimental.pallas import tpu as pltpu

_NUM_CLASSES = 80
_INPUT_SIZE = 608.0
_ANCHORS = np.array(
    [[10, 13], [16, 30], [33, 23], [30, 61], [62, 45], [59, 119],
     [116, 90], [156, 198], [373, 326]], dtype=np.float32)
_MASKS = [[6, 7, 8], [3, 4, 5], [0, 1, 2]]
_TOPK = 1000
_MAX_DET = 100
_IOU_TH = 0.5
_GRIDS = (19, 38, 76)
_N = 3 * sum(g * g for g in _GRIDS)
_ROWS = (_N + 127) // 128
_NPAD = _ROWS * 128
_C = 5 + _NUM_CLASSES


def _build_consts() -> np.ndarray:
    gxs, gys, gs, aws, ahs = [], [], [], [], []
    for G, m in zip(_GRIDS, _MASKS):
        cell = np.arange(G * G)
        gx = (cell % G).astype(np.float32)
        gy = (cell // G).astype(np.float32)
        gxs.append(np.repeat(gx, 3))
        gys.append(np.repeat(gy, 3))
        gs.append(np.full(G * G * 3, float(G), np.float32))
        anc = _ANCHORS[np.array(m)]
        aws.append(np.tile(anc[:, 0], G * G))
        ahs.append(np.tile(anc[:, 1], G * G))
    out = np.stack([np.concatenate(v) for v in (gxs, gys, gs, aws, ahs)])
    pad = np.ones((5, _NPAD - _N), np.float32)
    out = np.concatenate([out, pad], axis=1)
    return out.reshape(5, _ROWS, 128)


def _extract(arr, onehot):
    return jnp.sum(jnp.where(onehot, arr, 0.0))


def _yolo_kernel(const_ref, x_ref, out_ref):
    x = x_ref[0]
    gx = const_ref[0]
    gy = const_ref[1]
    gv = const_ref[2]
    aw = const_ref[3]
    ah = const_ref[4]

    obj = jax.nn.sigmoid(x[4])
    cls_s = jax.nn.sigmoid(x[5:])
    prod = cls_s * obj[None]
    conf = jnp.max(prod, axis=0)
    riota = jax.lax.broadcasted_iota(jnp.int32, prod.shape, 0)
    cls = jnp.min(jnp.where(prod == conf[None], riota, 1000), axis=0)
    cls = cls.astype(jnp.float32)

    sx = jax.nn.sigmoid(x[0])
    sy = jax.nn.sigmoid(x[1])
    w = jnp.exp(jnp.clip(x[2], -10.0, 10.0)) * aw / _INPUT_SIZE
    h = jnp.exp(jnp.clip(x[3], -10.0, 10.0)) * ah / _INPUT_SIZE
    xc = (sx + gx) / gv
    yc = (sy + gy) / gv
    x1 = xc - w * 0.5
    y1 = yc - h * 0.5
    x2 = xc + w * 0.5
    y2 = yc + h * 0.5

    rr = jax.lax.broadcasted_iota(jnp.int32, (_ROWS, 128), 0)
    cc = jax.lax.broadcasted_iota(jnp.int32, (_ROWS, 128), 1)
    n = rr * 128 + cc
    conf = jnp.where(n < _N, conf, -1.0)

    bits = jax.lax.bitcast_convert_type(conf, jnp.int32)

    def bs_body(_, carry):
        lo, hi = carry
        mid = (lo + hi) // 2
        cnt = jnp.sum((bits >= mid).astype(jnp.int32))
        return (jnp.where(cnt >= _TOPK, mid, lo),
                jnp.where(cnt >= _TOPK, hi, mid))

    lo, hi = jax.lax.fori_loop(
        0, 31, bs_body, (jnp.int32(0), jnp.int32(0x40000000)))
    vbits = lo
    cnt_gt = jnp.sum((bits > vbits).astype(jnp.int32))
    need = _TOPK - cnt_gt
    eq = bits == vbits

    def ts_body(_, carry):
        lo_t, hi_t = carry
        mid = (lo_t + hi_t) // 2
        cnt = jnp.sum((eq & (n <= mid)).astype(jnp.int32))
        return (jnp.where(cnt >= need, lo_t, mid),
                jnp.where(cnt >= need, mid, hi_t))

    _, t_idx = jax.lax.fori_loop(
        0, 16, ts_body, (jnp.int32(-1), jnp.int32(_NPAD)))
    active0 = ((bits > vbits) | (eq & (n <= t_idx))).astype(jnp.float32)

    maxall = jnp.max(conf)
    j0 = jnp.min(jnp.where(conf == maxall, n, _NPAD))
    oh0 = n == j0
    t1x1 = _extract(x1, oh0)
    t1y1 = _extract(y1, oh0)
    t1x2 = _extract(x2, oh0)
    t1y2 = _extract(y2, oh0)
    t1cl = _extract(cls, oh0)

    area2 = jnp.clip(x2 - x1, 0.0) * jnp.clip(y2 - y1, 0.0)
    lane = jax.lax.broadcasted_iota(jnp.int32, (1, 128), 1)

    def nms_body(i, active):
        masked = jnp.where(active > 0, conf, -1e9)
        mv = jnp.max(masked)
        j = jnp.min(jnp.where(masked == mv, n, _NPAD))
        oh = n == j
        bx1 = _extract(x1, oh)
        by1 = _extract(y1, oh)
        bx2 = _extract(x2, oh)
        by2 = _extract(y2, oh)
        bcl = _extract(cls, oh)
        ex = mv <= -1e8

        fx1 = jnp.where(ex, t1x1, bx1)
        fy1 = jnp.where(ex, t1y1, by1)
        fx2 = jnp.where(ex, t1x2, bx2)
        fy2 = jnp.where(ex, t1y2, by2)
        fcl = jnp.where(ex, t1cl, bcl)
        fsc = jnp.where(ex, 0.0, mv)
        row = jnp.where(lane == 0, fx1,
              jnp.where(lane == 1, fy1,
              jnp.where(lane == 2, fx2,
              jnp.where(lane == 3, fy2,
              jnp.where(lane == 4, fsc,
              jnp.where(lane == 5, fcl, 0.0))))))
        out_ref[0, pl.ds(i, 1), :] = row

        ix1 = jnp.maximum(bx1, x1)
        iy1 = jnp.maximum(by1, y1)
        ix2 = jnp.minimum(bx2, x2)
        iy2 = jnp.minimum(by2, y2)
        inter = jnp.clip(ix2 - ix1, 0.0) * jnp.clip(iy2 - iy1, 0.0)
        area1 = jnp.clip(bx2 - bx1, 0.0) * jnp.clip(by2 - by1, 0.0)
        iou = inter / (area1 + area2 - inter + 1e-9)
        active = jnp.where(iou > _IOU_TH, 0.0, active)
        active = jnp.where(oh, 0.0, active)
        return active

    jax.lax.fori_loop(0, _MAX_DET, nms_body, active0)


def kernel(inputs_0, inputs_1, inputs_2):
    B = inputs_0.shape[0]
    parts = []
    for x in (inputs_0, inputs_1, inputs_2):
        G = x.shape[1]
        parts.append(jnp.transpose(x.reshape(B, G * G * 3, _C), (0, 2, 1)))
    full = jnp.concatenate(parts, axis=2)
    full = jnp.pad(full, ((0, 0), (0, 0), (0, _NPAD - _N)))
    full = full.reshape(B, _C, _ROWS, 128)
    consts = jnp.asarray(_build_consts())

    out = pl.pallas_call(
        _yolo_kernel,
        grid=(B,),
        in_specs=[
            pl.BlockSpec((5, _ROWS, 128), lambda b: (0, 0, 0)),
            pl.BlockSpec((1, _C, _ROWS, 128), lambda b: (b, 0, 0, 0)),
        ],
        out_specs=pl.BlockSpec((1, _MAX_DET, 128), lambda b: (b, 0, 0)),
        out_shape=jax.ShapeDtypeStruct((B, _MAX_DET, 128), jnp.float32),
        compiler_params=pltpu.CompilerParams(
            dimension_semantics=("arbitrary",)),
    )(consts, full)
    return out[:, :, :6]
